# baseline (speedup 1.0000x reference)
import jax
import jax.numpy as jnp
from jax import lax
from jax.experimental import pallas as pl
from jax.experimental.pallas import tpu as pltpu
from jax.experimental.pallas import tpu_sc as plsc

ROWS, COLS = 4096, 8192
NC, NS, L = 2, 16, 16
NW = NC * NS
ROWS_PER_TILE = ROWS // NW
RIN = 4
ROUT = 2
NBIN = ROWS_PER_TILE // RIN
UNROLL = 4
NGROUPS = COLS // L


def _gather_batch4(idx_v, in_ref, out0, out1):
    @plsc.parallel_loop(0, NGROUPS, unroll=UNROLL)
    def g_body(g):
        o = g * L
        idx = idx_v[pl.ds(o, L)]
        for r in range(RIN):
            row_idx = jnp.full((L,), r, dtype=jnp.int32)
            dst = out0 if r < ROUT else out1
            dst[r % ROUT, pl.ds(o, L)] = plsc.load_gather(in_ref, [row_idx, idx])


def _permute_body(tensor_hbm, perm_hbm, out_hbm, idx_v,
                  in0, in1, out0, out1,
                  si0, si1, so0, so1):
    wid = lax.axis_index("s") * NC + lax.axis_index("c")
    base = wid * ROWS_PER_TILE
    pltpu.sync_copy(perm_hbm, idx_v)

    ins = (in0, in1)
    sins = (si0, si1)
    outs = (out0, out1)
    souts = (so0, so1)

    def start_in(b, s):
        pltpu.make_async_copy(
            tensor_hbm.at[pl.ds(base + b * RIN, RIN)], ins[s], sins[s]
        ).start()

    def wait_in(b, s):
        pltpu.make_async_copy(
            tensor_hbm.at[pl.ds(base + b * RIN, RIN)], ins[s], sins[s]
        ).wait()

    def start_out(ob, s):
        pltpu.make_async_copy(
            outs[s], out_hbm.at[pl.ds(base + ob * ROUT, ROUT)], souts[s]
        ).start()

    def wait_out(ob, s):
        pltpu.make_async_copy(
            outs[s], out_hbm.at[pl.ds(base + ob * ROUT, ROUT)], souts[s]
        ).wait()

    start_in(0, 0)
    start_in(1, 1)

    def pair_body(i, c):
        for s in range(2):
            b = 2 * i + s
            wait_in(b, s)
            pl.when(b >= 1)(lambda: wait_out(2 * b - 2, 0))
            pl.when(b >= 1)(lambda: wait_out(2 * b - 1, 1))
            _gather_batch4(idx_v, ins[s], outs[0], outs[1])
            start_out(2 * b, 0)
            start_out(2 * b + 1, 1)
            pl.when(b + 2 < NBIN)(lambda: start_in(b + 2, s))
        return c

    lax.fori_loop(0, NBIN // 2, pair_body, 0, unroll=False)

    wait_out(2 * NBIN - 2, 0)
    wait_out(2 * NBIN - 1, 1)


def kernel(tensor, permute):
    perm32 = permute.astype(jnp.int32)
    mesh = plsc.VectorSubcoreMesh(core_axis_name="c", subcore_axis_name="s")
    f = pl.kernel(
        _permute_body,
        out_type=jax.ShapeDtypeStruct((ROWS, COLS), jnp.float32),
        mesh=mesh,
        scratch_types=[
            pltpu.VMEM((COLS,), jnp.int32),
            pltpu.VMEM((RIN, COLS), jnp.float32),
            pltpu.VMEM((RIN, COLS), jnp.float32),
            pltpu.VMEM((ROUT, COLS), jnp.float32),
            pltpu.VMEM((ROUT, COLS), jnp.float32),
            pltpu.SemaphoreType.DMA,
            pltpu.SemaphoreType.DMA,
            pltpu.SemaphoreType.DMA,
            pltpu.SemaphoreType.DMA,
        ],
        compiler_params=pltpu.CompilerParams(needs_layout_passes=False),
    )
    return f(tensor, perm32)


# speedup vs baseline: 1.1058x; 1.1058x over previous
import jax
import jax.numpy as jnp
from jax import lax
from jax.experimental import pallas as pl
from jax.experimental.pallas import tpu as pltpu
from jax.experimental.pallas import tpu_sc as plsc

ROWS, COLS = 4096, 8192
NC, NS, L = 2, 16, 16
NW = NC * NS
ROWS_PER_TILE = ROWS // NW
R = 2
NB = ROWS_PER_TILE // R
NIN = 4
NOUT = 2
UNROLL = 4
NGROUPS = COLS // L


def _gather_batch(idx_v, in_ref, out_ref):
    @plsc.parallel_loop(0, NGROUPS, unroll=UNROLL)
    def g_body(g):
        o = g * L
        idx = idx_v[pl.ds(o, L)]
        for r in range(R):
            row_idx = jnp.full((L,), r, dtype=jnp.int32)
            out_ref[r, pl.ds(o, L)] = plsc.load_gather(in_ref, [row_idx, idx])


def _permute_body(tensor_hbm, perm_hbm, out_hbm, idx_v,
                  in0, in1, in2, in3, out0, out1,
                  si0, si1, si2, si3, so0, so1):
    wid = lax.axis_index("s") * NC + lax.axis_index("c")
    base = wid * ROWS_PER_TILE
    pltpu.sync_copy(perm_hbm, idx_v)

    ins = (in0, in1, in2, in3)
    sins = (si0, si1, si2, si3)
    outs = (out0, out1)
    souts = (so0, so1)

    def start_in(b, s):
        pltpu.make_async_copy(
            tensor_hbm.at[pl.ds(base + b * R, R)], ins[s], sins[s]
        ).start()

    def wait_in(b, s):
        pltpu.make_async_copy(
            tensor_hbm.at[pl.ds(base + b * R, R)], ins[s], sins[s]
        ).wait()

    def start_out(b, s):
        pltpu.make_async_copy(
            outs[s], out_hbm.at[pl.ds(base + b * R, R)], souts[s]
        ).start()

    def wait_out(b, s):
        pltpu.make_async_copy(
            outs[s], out_hbm.at[pl.ds(base + b * R, R)], souts[s]
        ).wait()

    for s in range(NIN):
        start_in(s, s)

    def quad_body(i, c):
        for s in range(NIN):
            b = NIN * i + s
            so = s % NOUT
            wait_in(b, s)
            pl.when(b >= NOUT)(lambda: wait_out(b - NOUT, so))
            _gather_batch(idx_v, ins[s], outs[so])
            start_out(b, so)
            pl.when(b + NIN < NB)(lambda: start_in(b + NIN, s))
        return c

    lax.fori_loop(0, NB // NIN, quad_body, 0, unroll=False)

    wait_out(NB - 2, 0)
    wait_out(NB - 1, 1)


def kernel(tensor, permute):
    perm32 = permute.astype(jnp.int32)
    mesh = plsc.VectorSubcoreMesh(core_axis_name="c", subcore_axis_name="s")
    f = pl.kernel(
        _permute_body,
        out_type=jax.ShapeDtypeStruct((ROWS, COLS), jnp.float32),
        mesh=mesh,
        scratch_types=[
            pltpu.VMEM((COLS,), jnp.int32),
            pltpu.VMEM((R, COLS), jnp.float32),
            pltpu.VMEM((R, COLS), jnp.float32),
            pltpu.VMEM((R, COLS), jnp.float32),
            pltpu.VMEM((R, COLS), jnp.float32),
            pltpu.VMEM((R, COLS), jnp.float32),
            pltpu.VMEM((R, COLS), jnp.float32),
            pltpu.SemaphoreType.DMA,
            pltpu.SemaphoreType.DMA,
            pltpu.SemaphoreType.DMA,
            pltpu.SemaphoreType.DMA,
            pltpu.SemaphoreType.DMA,
            pltpu.SemaphoreType.DMA,
        ],
        compiler_params=pltpu.CompilerParams(needs_layout_passes=False),
    )
    return f(tensor, perm32)
